# baseline (device time: 60250 ns/iter reference)
import jax
import jax.numpy as jnp
from jax import lax
from jax.experimental import pallas as pl
from jax.experimental.pallas import tpu as pltpu

N_DEV = 4
NCK = 4
D_ORDER = (1, 3, 2)
XH_ORDER = (1, 1, 3, 3, 2, 2, 0, 0)
Q_SLOT = {1: 0, 3: 1}
BF_SLOT = {0: 0, 2: 1}


def kernel(x, w_mat):
    m_tot, k_per = x.shape
    k_tot, n = w_mat.shape
    m_per = m_tot // N_DEV
    m_hf = m_per // 2
    m_ck = m_per // NCK
    k_hf = k_per // 2

    def body(x_hbm, w_hbm, out_hbm, xstage, xbf, xqs, xgbf, xgq, wstage,
             wbb, acc, sscale, rscale, max_ref, xdma_sems, wdma_sems,
             odma_sems, send_sems, recv_sems, ssc_sems, rsc_sems,
             msend_sems, mrecv_sems):
        my = lax.axis_index("i")

        def xdma(h):
            d = XH_ORDER[h]
            blk = lax.rem(my + d, N_DEV)
            return pltpu.make_async_copy(
                x_hbm.at[pl.ds(blk * m_per + (h % 2) * m_hf, m_hf), :],
                xstage.at[h % 2],
                xdma_sems.at[h % 2],
            )

        w_d = (0,) + D_ORDER

        def wdma(t, half):
            blk = lax.rem(my + (N_DEV - w_d[t]), N_DEV)
            rows = pl.ds(blk * k_per + half * k_hf, k_hf)
            return pltpu.make_async_copy(
                w_hbm.at[rows, :],
                wstage.at[t % 2, pl.ds(half * k_hf, k_hf), :],
                wdma_sems.at[t % 2, half],
            )

        xdma(0).start()
        xdma(1).start()

        barrier_sem = pltpu.get_barrier_semaphore()
        for d in range(1, N_DEV):
            peer = lax.rem(my + d, N_DEV)
            pl.semaphore_signal(
                barrier_sem, inc=1,
                device_id=(peer,), device_id_type=pl.DeviceIdType.MESH,
            )
        pl.semaphore_wait(barrier_sem, N_DEV - 1)

        def a2a_q(d, c):
            rows = pl.ds(c * m_ck, m_ck)
            s = Q_SLOT[d]
            return pltpu.make_async_remote_copy(
                src_ref=xqs.at[s, rows, :],
                dst_ref=xgq.at[s, rows, :],
                send_sem=send_sems.at[d, c],
                recv_sem=recv_sems.at[d, c],
                device_id=(lax.rem(my + d, N_DEV),),
                device_id_type=pl.DeviceIdType.MESH,
            )

        def a2a_bf(c):
            rows = pl.ds(c * m_ck, m_ck)
            return pltpu.make_async_remote_copy(
                src_ref=xbf.at[1, rows, :],
                dst_ref=xgbf.at[rows, :],
                send_sem=send_sems.at[2, c],
                recv_sem=recv_sems.at[2, c],
                device_id=(lax.rem(my + 2, N_DEV),),
                device_id_type=pl.DeviceIdType.MESH,
            )

        def scale_rdma(d):
            s = Q_SLOT[d]
            return pltpu.make_async_remote_copy(
                src_ref=sscale.at[s],
                dst_ref=rscale.at[s],
                send_sem=ssc_sems.at[s],
                recv_sem=rsc_sems.at[s],
                device_id=(lax.rem(my + d, N_DEV),),
                device_id_type=pl.DeviceIdType.MESH,
            )

        for h in range(8):
            xdma(h).wait()
            d = XH_ORDER[h]
            half_rows = pl.ds((h % 2) * m_hf, m_hf)
            if d in (1, 3):
                qs = Q_SLOT[d]
                for c in range(2 * (h % 2), 2 * (h % 2) + 2):
                    lrows = pl.ds((c % 2) * m_ck, m_ck)
                    grows = pl.ds(c * m_ck, m_ck)
                    a = xstage[h % 2, lrows, :]
                    amax = jnp.max(jnp.abs(a))
                    inv = 127.0 / amax
                    xqs[qs, grows, :] = jnp.round(a * inv).astype(jnp.int8)
                    sscale[qs, pl.ds(c, 1), :] = jnp.full(
                        (1, 128), amax / 127.0, jnp.float32
                    )
                    a2a_q(d, c).start()
                if h % 2 == 1:
                    scale_rdma(d).start()
            else:
                xbf[BF_SLOT[d], half_rows, :] = (
                    xstage[h % 2].astype(jnp.bfloat16)
                )
                if d == 2:
                    for c in range(2 * (h % 2), 2 * (h % 2) + 2):
                        a2a_bf(c).start()
            if h + 2 < 8:
                xdma(h + 2).start()
            if h == 5:
                wdma(0, 0).start()
                wdma(0, 1).start()
            if h == 7:
                wdma(1, 0).start()
                wdma(1, 1).start()

        local_max = jnp.float32(0.0)
        for t in range(4):
            d = w_d[t]
            wdma(t, 0).wait()
            wdma(t, 1).wait()
            wbb[t % 2] = wstage[t % 2].astype(jnp.bfloat16)
            if t + 2 < 4:
                wdma(t + 2, 0).start()
                wdma(t + 2, 1).start()
            if t in (1, 2):
                scale_rdma(d).wait_recv()

            for c in range(NCK):
                rows = pl.ds(c * m_ck, m_ck)
                if t == 0:
                    x_blk = xbf[0, rows, :]
                elif t < 3:
                    a2a_q(d, c).wait_recv()
                    qs = Q_SLOT[d]
                    x_blk = (
                        xgq[qs, rows, :].astype(jnp.float32)
                        * rscale[qs, c, 0]
                    ).astype(jnp.bfloat16)
                else:
                    a2a_bf(c).wait_recv()
                    x_blk = xgbf[rows, :]
                p = jnp.dot(x_blk, wbb[t % 2],
                            preferred_element_type=jnp.float32)
                if t == 0:
                    acc[rows, :] = p
                elif t < 3:
                    acc[rows, :] += p
                else:
                    r = jnp.maximum(acc[rows, :] + p, 0.0)
                    acc[rows, :] = r
                    local_max = jnp.maximum(local_max, jnp.max(r))

        max_ref[0, :, :] = jnp.full((8, 128), local_max, jnp.float32)

        mrdmas = []
        for d in range(1, N_DEV):
            peer = lax.rem(my + d, N_DEV)
            r = pltpu.make_async_remote_copy(
                src_ref=max_ref.at[0],
                dst_ref=max_ref.at[d],
                send_sem=msend_sems.at[d],
                recv_sem=mrecv_sems.at[d],
                device_id=(peer,),
                device_id_type=pl.DeviceIdType.MESH,
            )
            r.start()
            mrdmas.append(r)
        for r in mrdmas:
            r.wait_recv()

        gmax = jnp.max(max_ref[:, 0, 0])
        inv_scale = 127.0 / gmax
        scale = gmax / 127.0
        odmas = []
        for c in range(NCK):
            rows = pl.ds(c * m_ck, m_ck)
            q = jnp.clip(jnp.round(acc[rows, :] * inv_scale),
                         -127.0, 127.0)
            acc[rows, :] = q * scale
            o = pltpu.make_async_copy(
                acc.at[rows, :], out_hbm.at[rows, :], odma_sems.at[c]
            )
            o.start()
            odmas.append(o)

        for o in odmas:
            o.wait()
        for c in range(NCK):
            a2a_q(1, c).wait_send()
            a2a_q(3, c).wait_send()
            a2a_bf(c).wait_send()
        scale_rdma(1).wait_send()
        scale_rdma(3).wait_send()
        for r in mrdmas:
            r.wait_send()

    return pl.pallas_call(
        body,
        out_shape=jax.ShapeDtypeStruct((m_per, n), jnp.float32),
        in_specs=[
            pl.BlockSpec(memory_space=pl.ANY),
            pl.BlockSpec(memory_space=pl.ANY),
        ],
        out_specs=pl.BlockSpec(memory_space=pl.ANY),
        scratch_shapes=[
            pltpu.VMEM((2, m_hf, k_per), jnp.float32),
            pltpu.VMEM((2, m_per, k_per), jnp.bfloat16),
            pltpu.VMEM((2, m_per, k_per), jnp.int8),
            pltpu.VMEM((m_per, k_per), jnp.bfloat16),
            pltpu.VMEM((2, m_per, k_per), jnp.int8),
            pltpu.VMEM((2, k_per, n), jnp.float32),
            pltpu.VMEM((2, k_per, n), jnp.bfloat16),
            pltpu.VMEM((m_per, n), jnp.float32),
            pltpu.VMEM((2, 8, 128), jnp.float32),
            pltpu.VMEM((2, 8, 128), jnp.float32),
            pltpu.VMEM((N_DEV, 8, 128), jnp.float32),
            pltpu.SemaphoreType.DMA((2,)),
            pltpu.SemaphoreType.DMA((2, 2)),
            pltpu.SemaphoreType.DMA((NCK,)),
            pltpu.SemaphoreType.DMA((N_DEV, NCK)),
            pltpu.SemaphoreType.DMA((N_DEV, NCK)),
            pltpu.SemaphoreType.DMA((2,)),
            pltpu.SemaphoreType.DMA((2,)),
            pltpu.SemaphoreType.DMA((N_DEV,)),
            pltpu.SemaphoreType.DMA((N_DEV,)),
        ],
        compiler_params=pltpu.CompilerParams(
            collective_id=0,
            vmem_limit_bytes=60 * 1024 * 1024,
        ),
    )(x, w_mat)
